# baseline (speedup 1.0000x reference)
import functools

import jax
import jax.numpy as jnp
from jax import lax
from jax.experimental import pallas as pl
from jax.experimental.pallas import tpu as pltpu
from jax.experimental.pallas import tpu_sc as plsc

T = 2048
D = 1024
DFF = 4096
E = 8
TOPK = 2

BLK = 256
NPAD = T * TOPK + E * BLK
NB = NPAD // BLK

_info = plsc.get_sparse_core_info()
_NC, _NS = _info.num_cores, _info.num_subcores
NW = _NC * _NS

GCHUNK = 64
GROWS = NPAD // NW
CCHUNK = 32
TOKW = T // NW


def _sc_gather(src_token3d, input_batch):
    mesh = plsc.VectorSubcoreMesh(core_axis_name="c", subcore_axis_name="s")
    nchunk = GROWS // GCHUNK

    @functools.partial(
        pl.kernel, mesh=mesh,
        out_type=jax.ShapeDtypeStruct((NPAD, D), jnp.float32),
        scratch_types=[
            pltpu.VMEM((nchunk, GCHUNK), jnp.int32),
            pltpu.VMEM((GCHUNK, D), jnp.float32),
            pltpu.SemaphoreType.DMA,
        ],
    )
    def k(tok_hbm, inp_hbm, out_hbm, idx_v, rows_v, sem):
        wid = lax.axis_index("s") * _NC + lax.axis_index("c")
        pltpu.sync_copy(tok_hbm.at[wid], idx_v)
        for c in range(nchunk):
            pltpu.async_copy(inp_hbm.at[idx_v.at[c]], rows_v, sem).wait()
            pltpu.sync_copy(
                rows_v, out_hbm.at[pl.ds(wid * GROWS + c * GCHUNK, GCHUNK)])

    return k(src_token3d, input_batch)


def _tc_expert_mlp(block_expert, x_sorted, prob_col, w1, w2):
    def body(be_ref, x_ref, p_ref, w1_ref, w2_ref, out_ref):
        xs = (x_ref[...] * p_ref[...]).astype(jnp.bfloat16)
        h = jnp.dot(xs, w1_ref[0], preferred_element_type=jnp.float32)
        h = jnp.maximum(h, 0.0).astype(jnp.bfloat16)
        out_ref[...] = jnp.dot(h, w2_ref[0], preferred_element_type=jnp.float32)

    grid_spec = pltpu.PrefetchScalarGridSpec(
        num_scalar_prefetch=1,
        grid=(NB,),
        in_specs=[
            pl.BlockSpec((BLK, D), lambda i, be: (i, 0)),
            pl.BlockSpec((BLK, 1), lambda i, be: (i, 0)),
            pl.BlockSpec((1, D, DFF), lambda i, be: (be[i], 0, 0)),
            pl.BlockSpec((1, DFF, D), lambda i, be: (be[i], 0, 0)),
        ],
        out_specs=pl.BlockSpec((BLK, D), lambda i, be: (i, 0)),
    )
    return pl.pallas_call(
        body,
        grid_spec=grid_spec,
        out_shape=jax.ShapeDtypeStruct((NPAD, D), jnp.float32),
    )(block_expert, x_sorted, prob_col, w1, w2)


def _sc_combine(y_sorted, dest_a2d, dest_b2d):
    mesh = plsc.VectorSubcoreMesh(core_axis_name="c", subcore_axis_name="s")
    nchunk = TOKW // CCHUNK

    @functools.partial(
        pl.kernel, mesh=mesh,
        out_type=jax.ShapeDtypeStruct((T, D), jnp.float32),
        scratch_types=[
            pltpu.VMEM((nchunk, CCHUNK), jnp.int32),
            pltpu.VMEM((nchunk, CCHUNK), jnp.int32),
            pltpu.VMEM((CCHUNK, D), jnp.float32),
            pltpu.VMEM((CCHUNK, D), jnp.float32),
            pltpu.SemaphoreType.DMA,
            pltpu.SemaphoreType.DMA,
        ],
    )
    def k(y_hbm, da_hbm, db_hbm, out_hbm, ia_v, ib_v, ra_v, rb_v, sa, sb):
        wid = lax.axis_index("s") * _NC + lax.axis_index("c")
        pltpu.sync_copy(da_hbm.at[wid], ia_v)
        pltpu.sync_copy(db_hbm.at[wid], ib_v)
        for c in range(nchunk):
            cpa = pltpu.async_copy(y_hbm.at[ia_v.at[c]], ra_v, sa)
            cpb = pltpu.async_copy(y_hbm.at[ib_v.at[c]], rb_v, sb)
            cpa.wait()
            cpb.wait()

            def add_row(r, _):
                for q in range(D // 16):
                    sl = pl.ds(q * 16, 16)
                    ra_v[r, sl] = ra_v[r, sl] + rb_v[r, sl]
                return 0

            lax.fori_loop(0, CCHUNK, add_row, 0)
            pltpu.sync_copy(
                ra_v, out_hbm.at[pl.ds(wid * TOKW + c * CCHUNK, CCHUNK)])

    return k(y_sorted, dest_a2d, dest_b2d)


def kernel(input_batch, probabilities, indices, W1, W2):
    flat_e = indices.reshape(-1).astype(jnp.int32)
    flat_p = probabilities.reshape(-1).astype(jnp.float32)
    tok = jnp.arange(T * TOPK, dtype=jnp.int32) // TOPK
    onehot = (flat_e[:, None] == jnp.arange(E, dtype=jnp.int32)[None, :])
    onehot = onehot.astype(jnp.int32)
    counts = onehot.sum(axis=0)
    aligned = ((counts + BLK - 1) // BLK) * BLK
    ends = jnp.cumsum(aligned)
    starts = ends - aligned
    rank = jnp.cumsum(onehot, axis=0) - 1
    rank_i = jnp.take_along_axis(rank, flat_e[:, None], axis=1)[:, 0]
    dest = starts[flat_e] + rank_i
    src_token = jnp.zeros((NPAD,), jnp.int32).at[dest].set(tok)
    row_prob = jnp.zeros((NPAD,), jnp.float32).at[dest].set(flat_p)
    block_expert = jnp.minimum(
        jnp.searchsorted(ends, jnp.arange(NB, dtype=jnp.int32) * BLK,
                         side="right"),
        E - 1).astype(jnp.int32)

    x_sorted = _sc_gather(src_token.reshape(NW, -1, GCHUNK), input_batch)
    y_sorted = _tc_expert_mlp(
        block_expert, x_sorted, row_prob[:, None],
        W1.astype(jnp.bfloat16), W2.astype(jnp.bfloat16))
    d2 = dest.reshape(T, TOPK)
    out = _sc_combine(
        y_sorted,
        d2[:, 0].reshape(NW, -1, CCHUNK),
        d2[:, 1].reshape(NW, -1, CCHUNK))

    total_loss = jnp.zeros((), dtype=jnp.float32)
    return (out, total_loss)


# speedup vs baseline: 1.2880x; 1.2880x over previous
import functools

import jax
import jax.numpy as jnp
from jax import lax
from jax.experimental import pallas as pl
from jax.experimental.pallas import tpu as pltpu
from jax.experimental.pallas import tpu_sc as plsc

T = 2048
D = 1024
DFF = 4096
E = 8
TOPK = 2

BLK = 256
NPAD = T * TOPK + E * BLK
NB = NPAD // BLK

_info = plsc.get_sparse_core_info()
_NC, _NS = _info.num_cores, _info.num_subcores
NW = _NC * _NS

GCHUNK = 64
GROWS = NPAD // NW
CCHUNK = 32
TOKW = T // NW


def _sc_gather(src_token3d, input_batch):
    mesh = plsc.VectorSubcoreMesh(core_axis_name="c", subcore_axis_name="s")
    nchunk = GROWS // GCHUNK

    @functools.partial(
        pl.kernel, mesh=mesh,
        out_type=jax.ShapeDtypeStruct((NPAD, D), jnp.float32),
        scratch_types=[
            pltpu.VMEM((nchunk, GCHUNK), jnp.int32),
            pltpu.VMEM((GCHUNK, D), jnp.float32),
            pltpu.SemaphoreType.DMA,
        ],
    )
    def k(tok_hbm, inp_hbm, out_hbm, idx_v, rows_v, sem):
        wid = lax.axis_index("s") * _NC + lax.axis_index("c")
        pltpu.sync_copy(tok_hbm.at[wid], idx_v)
        for c in range(nchunk):
            pltpu.async_copy(inp_hbm.at[idx_v.at[c]], rows_v, sem).wait()
            pltpu.sync_copy(
                rows_v, out_hbm.at[pl.ds(wid * GROWS + c * GCHUNK, GCHUNK)])

    return k(src_token3d, input_batch)


def _tc_expert_mlp(block_expert, x_sorted, prob_col, w1, w2):
    def body(be_ref, x_ref, p_ref, w1_ref, w2_ref, out_ref):
        xs = (x_ref[...] * p_ref[...]).astype(jnp.bfloat16)
        h = jnp.dot(xs, w1_ref[0], preferred_element_type=jnp.float32)
        h = jnp.maximum(h, 0.0).astype(jnp.bfloat16)
        out_ref[...] = jnp.dot(h, w2_ref[0], preferred_element_type=jnp.float32)

    grid_spec = pltpu.PrefetchScalarGridSpec(
        num_scalar_prefetch=1,
        grid=(NB,),
        in_specs=[
            pl.BlockSpec((BLK, D), lambda i, be: (i, 0)),
            pl.BlockSpec((BLK, 1), lambda i, be: (i, 0)),
            pl.BlockSpec((1, D, DFF), lambda i, be: (be[i], 0, 0)),
            pl.BlockSpec((1, DFF, D), lambda i, be: (be[i], 0, 0)),
        ],
        out_specs=pl.BlockSpec((BLK, D), lambda i, be: (i, 0)),
    )
    return pl.pallas_call(
        body,
        grid_spec=grid_spec,
        out_shape=jax.ShapeDtypeStruct((NPAD, D), jnp.float32),
    )(block_expert, x_sorted, prob_col, w1, w2)


def _sc_combine(y_sorted, dest_a2d, dest_b2d):
    mesh = plsc.VectorSubcoreMesh(core_axis_name="c", subcore_axis_name="s")
    nchunk = TOKW // CCHUNK

    @functools.partial(
        pl.kernel, mesh=mesh,
        out_type=jax.ShapeDtypeStruct((T, D), jnp.float32),
        scratch_types=[
            pltpu.VMEM((nchunk, CCHUNK), jnp.int32),
            pltpu.VMEM((nchunk, CCHUNK), jnp.int32),
            pltpu.VMEM((CCHUNK, D), jnp.float32),
            pltpu.VMEM((CCHUNK, D), jnp.float32),
            pltpu.SemaphoreType.DMA,
            pltpu.SemaphoreType.DMA,
        ],
    )
    def k(y_hbm, da_hbm, db_hbm, out_hbm, ia_v, ib_v, ra_v, rb_v, sa, sb):
        wid = lax.axis_index("s") * _NC + lax.axis_index("c")
        pltpu.sync_copy(da_hbm.at[wid], ia_v)
        pltpu.sync_copy(db_hbm.at[wid], ib_v)
        for c in range(nchunk):
            cpa = pltpu.async_copy(y_hbm.at[ia_v.at[c]], ra_v, sa)
            cpb = pltpu.async_copy(y_hbm.at[ib_v.at[c]], rb_v, sb)
            cpa.wait()
            cpb.wait()

            def add_row(r, _):
                for q in range(D // 16):
                    sl = pl.ds(q * 16, 16)
                    ra_v[r, sl] = ra_v[r, sl] + rb_v[r, sl]
                return 0

            lax.fori_loop(0, CCHUNK, add_row, 0)
            pltpu.sync_copy(
                ra_v, out_hbm.at[pl.ds(wid * TOKW + c * CCHUNK, CCHUNK)])

    return k(y_sorted, dest_a2d, dest_b2d)


def kernel(input_batch, probabilities, indices, W1, W2):
    flat_e = indices.reshape(-1).astype(jnp.int32)
    flat_p = probabilities.reshape(-1).astype(jnp.float32)
    tok = jnp.arange(T * TOPK, dtype=jnp.int32) // TOPK
    onehot = (flat_e[:, None] == jnp.arange(E, dtype=jnp.int32)[None, :])
    onehot = onehot.astype(jnp.int32)
    counts = onehot.sum(axis=0)
    aligned = ((counts + BLK - 1) // BLK) * BLK
    ends = jnp.cumsum(aligned)
    starts = ends - aligned
    rank = jnp.cumsum(onehot, axis=0) - 1
    rank_i = jnp.take_along_axis(rank, flat_e[:, None], axis=1)[:, 0]
    dest = starts[flat_e] + rank_i
    pad_spread = jnp.arange(NPAD, dtype=jnp.int32) % T
    src_token = pad_spread.at[dest].set(tok)
    row_prob = jnp.zeros((NPAD,), jnp.float32).at[dest].set(flat_p)
    block_expert = jnp.minimum(
        jnp.searchsorted(ends, jnp.arange(NB, dtype=jnp.int32) * BLK,
                         side="right"),
        E - 1).astype(jnp.int32)

    x_sorted = _sc_gather(src_token.reshape(NW, -1, GCHUNK), input_batch)
    y_sorted = _tc_expert_mlp(
        block_expert, x_sorted, row_prob[:, None],
        W1.astype(jnp.bfloat16), W2.astype(jnp.bfloat16))
    d2 = dest.reshape(T, TOPK)
    out = _sc_combine(
        y_sorted,
        d2[:, 0].reshape(NW, -1, CCHUNK),
        d2[:, 1].reshape(NW, -1, CCHUNK))

    total_loss = jnp.zeros((), dtype=jnp.float32)
    return (out, total_loss)


# speedup vs baseline: 1.3321x; 1.0342x over previous
import functools

import jax
import jax.numpy as jnp
from jax import lax
from jax.experimental import pallas as pl
from jax.experimental.pallas import tpu as pltpu
from jax.experimental.pallas import tpu_sc as plsc

T = 2048
D = 1024
DFF = 4096
E = 8
TOPK = 2

BLK = 256
NPAD = T * TOPK + E * BLK
NB = NPAD // BLK

_info = plsc.get_sparse_core_info()
_NC, _NS = _info.num_cores, _info.num_subcores
NW = _NC * _NS

GCHUNK = 64
GROWS = NPAD // NW
CCHUNK = 32
TOKW = T // NW


def _sc_gather(src_token3d, input_batch):
    mesh = plsc.VectorSubcoreMesh(core_axis_name="c", subcore_axis_name="s")
    nchunk = GROWS // GCHUNK

    @functools.partial(
        pl.kernel, mesh=mesh,
        out_type=jax.ShapeDtypeStruct((NPAD, D), jnp.float32),
        scratch_types=[
            pltpu.VMEM((nchunk, GCHUNK), jnp.int32),
            pltpu.VMEM((GCHUNK, D), jnp.float32),
            pltpu.SemaphoreType.DMA,
        ],
    )
    def k(tok_hbm, inp_hbm, out_hbm, idx_v, rows_v, sem):
        wid = lax.axis_index("s") * _NC + lax.axis_index("c")
        pltpu.sync_copy(tok_hbm.at[wid], idx_v)
        for c in range(nchunk):
            pltpu.async_copy(inp_hbm.at[idx_v.at[c]], rows_v, sem).wait()
            pltpu.sync_copy(
                rows_v, out_hbm.at[pl.ds(wid * GROWS + c * GCHUNK, GCHUNK)])

    return k(src_token3d, input_batch)


def _tc_expert_mlp(block_expert, x_sorted, prob_col, w1, w2):
    def body(be_ref, x_ref, p_ref, w1_ref, w2_ref, out_ref):
        xs = (x_ref[...] * p_ref[...]).astype(jnp.bfloat16)
        h = jnp.dot(xs, w1_ref[0], preferred_element_type=jnp.float32)
        h = jnp.maximum(h, 0.0).astype(jnp.bfloat16)
        out_ref[...] = jnp.dot(h, w2_ref[0], preferred_element_type=jnp.float32)

    grid_spec = pltpu.PrefetchScalarGridSpec(
        num_scalar_prefetch=1,
        grid=(NB,),
        in_specs=[
            pl.BlockSpec((BLK, D), lambda i, be: (i, 0)),
            pl.BlockSpec((BLK, 1), lambda i, be: (i, 0)),
            pl.BlockSpec((1, D, DFF), lambda i, be: (be[i], 0, 0)),
            pl.BlockSpec((1, DFF, D), lambda i, be: (be[i], 0, 0)),
        ],
        out_specs=pl.BlockSpec((BLK, D), lambda i, be: (i, 0)),
    )
    return pl.pallas_call(
        body,
        grid_spec=grid_spec,
        out_shape=jax.ShapeDtypeStruct((NPAD, D), jnp.float32),
    )(block_expert, x_sorted, prob_col, w1, w2)


def _sc_combine(y_sorted, dest_a2d, dest_b2d):
    mesh = plsc.VectorSubcoreMesh(core_axis_name="c", subcore_axis_name="s")
    nchunk = TOKW // CCHUNK

    @functools.partial(
        pl.kernel, mesh=mesh,
        out_type=jax.ShapeDtypeStruct((T, D), jnp.float32),
        scratch_types=[
            pltpu.VMEM((nchunk, CCHUNK), jnp.int32),
            pltpu.VMEM((nchunk, CCHUNK), jnp.int32),
            pltpu.VMEM((CCHUNK, D), jnp.float32),
            pltpu.VMEM((CCHUNK, D), jnp.float32),
            pltpu.SemaphoreType.DMA,
            pltpu.SemaphoreType.DMA,
        ],
    )
    def k(y_hbm, da_hbm, db_hbm, out_hbm, ia_v, ib_v, ra_v, rb_v, sa, sb):
        wid = lax.axis_index("s") * _NC + lax.axis_index("c")
        pltpu.sync_copy(da_hbm.at[wid], ia_v)
        pltpu.sync_copy(db_hbm.at[wid], ib_v)
        for c in range(nchunk):
            cpa = pltpu.async_copy(y_hbm.at[ia_v.at[c]], ra_v, sa)
            cpb = pltpu.async_copy(y_hbm.at[ib_v.at[c]], rb_v, sb)
            cpa.wait()
            cpb.wait()

            def add_row(r, _):
                for q in range(D // 16):
                    sl = pl.ds(q * 16, 16)
                    ra_v[r, sl] = ra_v[r, sl] + rb_v[r, sl]
                return 0

            lax.fori_loop(0, CCHUNK, add_row, 0)
            pltpu.sync_copy(
                ra_v, out_hbm.at[pl.ds(wid * TOKW + c * CCHUNK, CCHUNK)])

    return k(y_sorted, dest_a2d, dest_b2d)


def kernel(input_batch, probabilities, indices, W1, W2):
    flat_e = indices.reshape(-1).astype(jnp.int32)
    tok = jnp.arange(T * TOPK, dtype=jnp.int32) // TOPK
    onehot = (flat_e[:, None] == jnp.arange(E, dtype=jnp.int32)[None, :])
    onehot = onehot.astype(jnp.int32)
    counts = onehot.sum(axis=0)
    aligned = ((counts + BLK - 1) // BLK) * BLK
    ends = jnp.cumsum(aligned)
    starts = ends - aligned
    rank = jnp.cumsum(onehot, axis=0) - 1
    rank_i = jnp.take_along_axis(rank, flat_e[:, None], axis=1)[:, 0]
    dest = starts[flat_e] + rank_i
    pad_spread = jnp.arange(NPAD, dtype=jnp.int32) % T
    flat_p = probabilities.reshape(-1).astype(jnp.float32)
    pairs = jnp.stack([tok, lax.bitcast_convert_type(flat_p, jnp.int32)],
                      axis=1)
    init = jnp.stack([pad_spread, jnp.zeros((NPAD,), jnp.int32)], axis=1)
    packed = init.at[dest].set(pairs)
    src_token = packed[:, 0]
    row_prob = lax.bitcast_convert_type(packed[:, 1], jnp.float32)
    block_expert = jnp.minimum(
        jnp.searchsorted(ends, jnp.arange(NB, dtype=jnp.int32) * BLK,
                         side="right"),
        E - 1).astype(jnp.int32)

    x_sorted = _sc_gather(src_token.reshape(NW, -1, GCHUNK), input_batch)
    y_sorted = _tc_expert_mlp(
        block_expert, x_sorted, row_prob[:, None],
        W1.astype(jnp.bfloat16), W2.astype(jnp.bfloat16))
    d2 = dest.reshape(T, TOPK)
    out = _sc_combine(
        y_sorted,
        d2[:, 0].reshape(NW, -1, CCHUNK),
        d2[:, 1].reshape(NW, -1, CCHUNK))

    total_loss = jnp.zeros((), dtype=jnp.float32)
    return (out, total_loss)


# speedup vs baseline: 1.4006x; 1.0514x over previous
import functools

import jax
import jax.numpy as jnp
from jax import lax
from jax.experimental import pallas as pl
from jax.experimental.pallas import tpu as pltpu
from jax.experimental.pallas import tpu_sc as plsc

T = 2048
D = 1024
DFF = 4096
E = 8
TOPK = 2

BLK = 256
NPAD = T * TOPK + E * BLK
NB = NPAD // BLK

_info = plsc.get_sparse_core_info()
_NC, _NS = _info.num_cores, _info.num_subcores
NW = _NC * _NS

GCHUNK = 64
GROWS = NPAD // NW
CCHUNK = 32
TOKW = T // NW


def _sc_gather(src_token3d, input_batch):
    mesh = plsc.VectorSubcoreMesh(core_axis_name="c", subcore_axis_name="s")
    nchunk = GROWS // GCHUNK

    @functools.partial(
        pl.kernel, mesh=mesh,
        out_type=jax.ShapeDtypeStruct((NPAD, D), jnp.float32),
        scratch_types=[
            pltpu.VMEM((nchunk, GCHUNK), jnp.int32),
            pltpu.VMEM((GCHUNK, D), jnp.float32),
            pltpu.SemaphoreType.DMA,
        ],
    )
    def k(tok_hbm, inp_hbm, out_hbm, idx_v, rows_v, sem):
        wid = lax.axis_index("s") * _NC + lax.axis_index("c")
        pltpu.sync_copy(tok_hbm.at[wid], idx_v)
        for c in range(nchunk):
            pltpu.async_copy(inp_hbm.at[idx_v.at[c]], rows_v, sem).wait()
            pltpu.sync_copy(
                rows_v, out_hbm.at[pl.ds(wid * GROWS + c * GCHUNK, GCHUNK)])

    return k(src_token3d, input_batch)


def _tc_expert_mlp(block_expert, x_sorted, prob_col, w1, w2):
    FJ = 2
    FB = DFF // FJ

    def body(be_ref, x_ref, p_ref, w1_ref, w2_ref, out_ref):
        j = pl.program_id(1)
        xs = (x_ref[...] * p_ref[...]).astype(jnp.bfloat16)
        h = jnp.dot(xs, w1_ref[0].astype(jnp.bfloat16),
                    preferred_element_type=jnp.float32)
        h = jnp.maximum(h, 0.0).astype(jnp.bfloat16)
        part = jnp.dot(h, w2_ref[0].astype(jnp.bfloat16),
                       preferred_element_type=jnp.float32)

        @pl.when(j == 0)
        def _():
            out_ref[...] = part

        @pl.when(j != 0)
        def _():
            out_ref[...] += part

    grid_spec = pltpu.PrefetchScalarGridSpec(
        num_scalar_prefetch=1,
        grid=(NB, FJ),
        in_specs=[
            pl.BlockSpec((BLK, D), lambda i, j, be: (i, 0)),
            pl.BlockSpec((BLK, 1), lambda i, j, be: (i, 0)),
            pl.BlockSpec((1, D, FB), lambda i, j, be: (be[i], 0, j)),
            pl.BlockSpec((1, FB, D), lambda i, j, be: (be[i], j, 0)),
        ],
        out_specs=pl.BlockSpec((BLK, D), lambda i, j, be: (i, 0)),
    )
    return pl.pallas_call(
        body,
        grid_spec=grid_spec,
        out_shape=jax.ShapeDtypeStruct((NPAD, D), jnp.float32),
    )(block_expert, x_sorted, prob_col, w1, w2)


def _sc_combine(y_sorted, dest_a2d, dest_b2d):
    mesh = plsc.VectorSubcoreMesh(core_axis_name="c", subcore_axis_name="s")
    nchunk = TOKW // CCHUNK

    @functools.partial(
        pl.kernel, mesh=mesh,
        out_type=jax.ShapeDtypeStruct((T, D), jnp.float32),
        scratch_types=[
            pltpu.VMEM((nchunk, CCHUNK), jnp.int32),
            pltpu.VMEM((nchunk, CCHUNK), jnp.int32),
            pltpu.VMEM((CCHUNK, D), jnp.float32),
            pltpu.VMEM((CCHUNK, D), jnp.float32),
            pltpu.SemaphoreType.DMA,
            pltpu.SemaphoreType.DMA,
        ],
    )
    def k(y_hbm, da_hbm, db_hbm, out_hbm, ia_v, ib_v, ra_v, rb_v, sa, sb):
        wid = lax.axis_index("s") * _NC + lax.axis_index("c")
        pltpu.sync_copy(da_hbm.at[wid], ia_v)
        pltpu.sync_copy(db_hbm.at[wid], ib_v)
        for c in range(nchunk):
            cpa = pltpu.async_copy(y_hbm.at[ia_v.at[c]], ra_v, sa)
            cpb = pltpu.async_copy(y_hbm.at[ib_v.at[c]], rb_v, sb)
            cpa.wait()
            cpb.wait()

            def add_row(r, _):
                for q in range(D // 16):
                    sl = pl.ds(q * 16, 16)
                    ra_v[r, sl] = ra_v[r, sl] + rb_v[r, sl]
                return 0

            lax.fori_loop(0, CCHUNK, add_row, 0)
            pltpu.sync_copy(
                ra_v, out_hbm.at[pl.ds(wid * TOKW + c * CCHUNK, CCHUNK)])

    return k(y_sorted, dest_a2d, dest_b2d)


def kernel(input_batch, probabilities, indices, W1, W2):
    flat_e = indices.reshape(-1).astype(jnp.int32)
    tok = jnp.arange(T * TOPK, dtype=jnp.int32) // TOPK
    onehot = (flat_e[:, None] == jnp.arange(E, dtype=jnp.int32)[None, :])
    onehot = onehot.astype(jnp.int32)
    counts = onehot.sum(axis=0)
    aligned = ((counts + BLK - 1) // BLK) * BLK
    ends = jnp.cumsum(aligned)
    starts = ends - aligned
    csum = jnp.cumsum(onehot, axis=0)
    rank_i = (csum * onehot).sum(axis=1) - 1
    start_i = (starts[None, :] * onehot).sum(axis=1)
    dest = start_i + rank_i
    pad_spread = jnp.arange(NPAD, dtype=jnp.int32) % T
    flat_p = probabilities.reshape(-1).astype(jnp.float32)
    pairs = jnp.stack([tok, lax.bitcast_convert_type(flat_p, jnp.int32)],
                      axis=1)
    init = jnp.stack([pad_spread, jnp.zeros((NPAD,), jnp.int32)], axis=1)
    packed = init.at[dest].set(pairs)
    src_token = packed[:, 0]
    row_prob = lax.bitcast_convert_type(packed[:, 1], jnp.float32)
    block_base = jnp.arange(NB, dtype=jnp.int32) * BLK
    block_expert = jnp.minimum(
        (block_base[:, None] >= ends[None, :]).sum(axis=1), E - 1
    ).astype(jnp.int32)

    x_sorted = _sc_gather(src_token.reshape(NW, -1, GCHUNK), input_batch)
    y_sorted = _tc_expert_mlp(block_expert, x_sorted, row_prob[:, None],
                              W1, W2)
    d2 = dest.reshape(T, TOPK)
    out = _sc_combine(
        y_sorted,
        d2[:, 0].reshape(NW, -1, CCHUNK),
        d2[:, 1].reshape(NW, -1, CCHUNK))

    total_loss = jnp.zeros((), dtype=jnp.float32)
    return (out, total_loss)


# speedup vs baseline: 1.5217x; 1.0865x over previous
import functools

import jax
import jax.numpy as jnp
from jax import lax
from jax.experimental import pallas as pl
from jax.experimental.pallas import tpu as pltpu
from jax.experimental.pallas import tpu_sc as plsc

T = 2048
D = 1024
DFF = 4096
E = 8
TOPK = 2

BLK = 256
NPAD = T * TOPK + E * BLK
NB = NPAD // BLK

_info = plsc.get_sparse_core_info()
_NC, _NS = _info.num_cores, _info.num_subcores
NW = _NC * _NS

GCHUNK = 64
GROWS = NPAD // NW
CCHUNK = 32
TOKW = T // NW


def _sc_gather(src_token3d, input_batch):
    mesh = plsc.VectorSubcoreMesh(core_axis_name="c", subcore_axis_name="s")
    nchunk = GROWS // GCHUNK

    @functools.partial(
        pl.kernel, mesh=mesh,
        out_type=jax.ShapeDtypeStruct((NPAD, D), jnp.float32),
        scratch_types=[
            pltpu.VMEM((nchunk, GCHUNK), jnp.int32),
            pltpu.VMEM((GCHUNK, D), jnp.float32),
            pltpu.SemaphoreType.DMA,
        ],
    )
    def k(tok_hbm, inp_hbm, out_hbm, idx_v, rows_v, sem):
        wid = lax.axis_index("s") * _NC + lax.axis_index("c")
        pltpu.sync_copy(tok_hbm.at[wid], idx_v)
        for c in range(nchunk):
            pltpu.async_copy(inp_hbm.at[idx_v.at[c]], rows_v, sem).wait()
            pltpu.sync_copy(
                rows_v, out_hbm.at[pl.ds(wid * GROWS + c * GCHUNK, GCHUNK)])

    return k(src_token3d, input_batch)


def _tc_expert_mlp(block_expert, x_sorted, prob_col, w1, w2):
    FJ = 2
    FB = DFF // FJ

    def body(be_ref, x_ref, p_ref, w1_ref, w2_ref, out_ref):
        j = pl.program_id(1)
        xs = (x_ref[...] * p_ref[...]).astype(jnp.bfloat16)
        h = jnp.dot(xs, w1_ref[0].astype(jnp.bfloat16),
                    preferred_element_type=jnp.float32)
        h = jnp.maximum(h, 0.0).astype(jnp.bfloat16)
        part = jnp.dot(h, w2_ref[0].astype(jnp.bfloat16),
                       preferred_element_type=jnp.float32)

        @pl.when(j == 0)
        def _():
            out_ref[...] = part

        @pl.when(j != 0)
        def _():
            out_ref[...] += part

    def _jj(i, j):
        return jnp.where(i % 2 == 0, j, FJ - 1 - j)

    grid_spec = pltpu.PrefetchScalarGridSpec(
        num_scalar_prefetch=1,
        grid=(NB, FJ),
        in_specs=[
            pl.BlockSpec((BLK, D), lambda i, j, be: (i, 0)),
            pl.BlockSpec((BLK, 1), lambda i, j, be: (i, 0)),
            pl.BlockSpec((1, D, FB), lambda i, j, be: (be[i], 0, _jj(i, j))),
            pl.BlockSpec((1, FB, D), lambda i, j, be: (be[i], _jj(i, j), 0)),
        ],
        out_specs=pl.BlockSpec((BLK, D), lambda i, j, be: (i, 0)),
    )
    return pl.pallas_call(
        body,
        grid_spec=grid_spec,
        out_shape=jax.ShapeDtypeStruct((NPAD, D), jnp.float32),
    )(block_expert, x_sorted, prob_col, w1, w2)


def _sc_combine(y_sorted, dest_a2d, dest_b2d):
    mesh = plsc.VectorSubcoreMesh(core_axis_name="c", subcore_axis_name="s")
    nchunk = TOKW // CCHUNK

    @functools.partial(
        pl.kernel, mesh=mesh,
        out_type=jax.ShapeDtypeStruct((T, D), jnp.float32),
        scratch_types=[
            pltpu.VMEM((nchunk, CCHUNK), jnp.int32),
            pltpu.VMEM((nchunk, CCHUNK), jnp.int32),
            pltpu.VMEM((CCHUNK, D), jnp.float32),
            pltpu.VMEM((CCHUNK, D), jnp.float32),
            pltpu.SemaphoreType.DMA,
            pltpu.SemaphoreType.DMA,
        ],
    )
    def k(y_hbm, da_hbm, db_hbm, out_hbm, ia_v, ib_v, ra_v, rb_v, sa, sb):
        wid = lax.axis_index("s") * _NC + lax.axis_index("c")
        pltpu.sync_copy(da_hbm.at[wid], ia_v)
        pltpu.sync_copy(db_hbm.at[wid], ib_v)
        for c in range(nchunk):
            cpa = pltpu.async_copy(y_hbm.at[ia_v.at[c]], ra_v, sa)
            cpb = pltpu.async_copy(y_hbm.at[ib_v.at[c]], rb_v, sb)
            cpa.wait()
            cpb.wait()

            def add_row(r, _):
                for q in range(D // 16):
                    sl = pl.ds(q * 16, 16)
                    ra_v[r, sl] = ra_v[r, sl] + rb_v[r, sl]
                return 0

            lax.fori_loop(0, CCHUNK, add_row, 0)
            pltpu.sync_copy(
                ra_v, out_hbm.at[pl.ds(wid * TOKW + c * CCHUNK, CCHUNK)])

    return k(y_sorted, dest_a2d, dest_b2d)


def kernel(input_batch, probabilities, indices, W1, W2):
    flat_e = indices.reshape(-1).astype(jnp.int32)
    tok = jnp.arange(T * TOPK, dtype=jnp.int32) // TOPK
    onehot = (flat_e[:, None] == jnp.arange(E, dtype=jnp.int32)[None, :])
    onehot = onehot.astype(jnp.int32)
    counts = onehot.sum(axis=0)
    aligned = ((counts + BLK - 1) // BLK) * BLK
    ends = jnp.cumsum(aligned)
    starts = ends - aligned
    csum = jnp.cumsum(onehot, axis=0)
    rank_i = (csum * onehot).sum(axis=1) - 1
    start_i = (starts[None, :] * onehot).sum(axis=1)
    dest = start_i + rank_i
    pad_spread = jnp.arange(NPAD, dtype=jnp.int32) % T
    flat_p = probabilities.reshape(-1).astype(jnp.float32)
    pairs = jnp.stack([tok, lax.bitcast_convert_type(flat_p, jnp.int32)],
                      axis=1)
    init = jnp.stack([pad_spread, jnp.zeros((NPAD,), jnp.int32)], axis=1)
    packed = init.at[dest].set(pairs)
    src_token = packed[:, 0]
    row_prob = lax.bitcast_convert_type(packed[:, 1], jnp.float32)
    block_base = jnp.arange(NB, dtype=jnp.int32) * BLK
    block_expert = jnp.minimum(
        (block_base[:, None] >= ends[None, :]).sum(axis=1), E - 1
    ).astype(jnp.int32)

    x_sorted = _sc_gather(src_token.reshape(NW, -1, GCHUNK), input_batch)
    y_sorted = _tc_expert_mlp(block_expert, x_sorted, row_prob[:, None],
                              W1, W2)
    d2 = dest.reshape(T, TOPK)
    out = _sc_combine(
        y_sorted,
        d2[:, 0].reshape(NW, -1, CCHUNK),
        d2[:, 1].reshape(NW, -1, CCHUNK))

    total_loss = jnp.zeros((), dtype=jnp.float32)
    return (out, total_loss)


# speedup vs baseline: 1.7997x; 1.1827x over previous
import functools

import jax
import jax.numpy as jnp
from jax import lax
from jax.experimental import pallas as pl
from jax.experimental.pallas import tpu as pltpu
from jax.experimental.pallas import tpu_sc as plsc

T = 2048
D = 1024
DFF = 4096
E = 8
TOPK = 2

BLK = 512
NPAD = T * TOPK + E * BLK
NB = NPAD // BLK

_info = plsc.get_sparse_core_info()
_NC, _NS = _info.num_cores, _info.num_subcores
NW = _NC * _NS

GCHUNK = 64
GROWS = NPAD // NW
CCHUNK = 32
TOKW = T // NW


def _sc_gather(src_token3d, input_batch):
    mesh = plsc.VectorSubcoreMesh(core_axis_name="c", subcore_axis_name="s")
    nchunk = GROWS // GCHUNK

    @functools.partial(
        pl.kernel, mesh=mesh,
        out_type=jax.ShapeDtypeStruct((NPAD, D), jnp.float32),
        scratch_types=[
            pltpu.VMEM((nchunk, GCHUNK), jnp.int32),
            pltpu.VMEM((GCHUNK, D), jnp.float32),
            pltpu.SemaphoreType.DMA,
        ],
    )
    def k(tok_hbm, inp_hbm, out_hbm, idx_v, rows_v, sem):
        wid = lax.axis_index("s") * _NC + lax.axis_index("c")
        pltpu.sync_copy(tok_hbm.at[wid], idx_v)
        for c in range(nchunk):
            pltpu.async_copy(inp_hbm.at[idx_v.at[c]], rows_v, sem).wait()
            pltpu.sync_copy(
                rows_v, out_hbm.at[pl.ds(wid * GROWS + c * GCHUNK, GCHUNK)])

    return k(src_token3d, input_batch)


def _tc_expert_mlp(block_expert, block_active, x_sorted, prob_col, w1, w2):
    FJ = 2
    FB = DFF // FJ

    def body(be_ref, act_ref, x_ref, p_ref, w1_ref, w2_ref, out_ref):
        i = pl.program_id(0)
        j = pl.program_id(1)

        @pl.when(act_ref[i] == 1)
        def _():
            xs = (x_ref[...] * p_ref[...]).astype(jnp.bfloat16)
            h = jnp.dot(xs, w1_ref[0].astype(jnp.bfloat16),
                        preferred_element_type=jnp.float32)
            h = jnp.maximum(h, 0.0).astype(jnp.bfloat16)
            part = jnp.dot(h, w2_ref[0].astype(jnp.bfloat16),
                           preferred_element_type=jnp.float32)

            @pl.when(j == 0)
            def _():
                out_ref[...] = part

            @pl.when(j != 0)
            def _():
                out_ref[...] += part

    def _jj(i, j):
        return jnp.where(i % 2 == 0, j, FJ - 1 - j)

    grid_spec = pltpu.PrefetchScalarGridSpec(
        num_scalar_prefetch=2,
        grid=(NB, FJ),
        in_specs=[
            pl.BlockSpec((BLK, D), lambda i, j, be, act: (i, 0)),
            pl.BlockSpec((BLK, 1), lambda i, j, be, act: (i, 0)),
            pl.BlockSpec((1, D, FB),
                         lambda i, j, be, act: (be[i], 0, _jj(i, j))),
            pl.BlockSpec((1, FB, D),
                         lambda i, j, be, act: (be[i], _jj(i, j), 0)),
        ],
        out_specs=pl.BlockSpec((BLK, D), lambda i, j, be, act: (i, 0)),
    )
    return pl.pallas_call(
        body,
        grid_spec=grid_spec,
        out_shape=jax.ShapeDtypeStruct((NPAD, D), jnp.float32),
    )(block_expert, block_active, x_sorted, prob_col, w1, w2)


def _sc_combine(y_sorted, dest_a2d, dest_b2d):
    mesh = plsc.VectorSubcoreMesh(core_axis_name="c", subcore_axis_name="s")
    nchunk = TOKW // CCHUNK

    @functools.partial(
        pl.kernel, mesh=mesh,
        out_type=jax.ShapeDtypeStruct((T, D), jnp.float32),
        scratch_types=[
            pltpu.VMEM((nchunk, CCHUNK), jnp.int32),
            pltpu.VMEM((nchunk, CCHUNK), jnp.int32),
            pltpu.VMEM((CCHUNK, D), jnp.float32),
            pltpu.VMEM((CCHUNK, D), jnp.float32),
            pltpu.SemaphoreType.DMA,
            pltpu.SemaphoreType.DMA,
        ],
    )
    def k(y_hbm, da_hbm, db_hbm, out_hbm, ia_v, ib_v, ra_v, rb_v, sa, sb):
        wid = lax.axis_index("s") * _NC + lax.axis_index("c")
        pltpu.sync_copy(da_hbm.at[wid], ia_v)
        pltpu.sync_copy(db_hbm.at[wid], ib_v)
        for c in range(nchunk):
            cpa = pltpu.async_copy(y_hbm.at[ia_v.at[c]], ra_v, sa)
            cpb = pltpu.async_copy(y_hbm.at[ib_v.at[c]], rb_v, sb)
            cpa.wait()
            cpb.wait()

            def add_row(r, _):
                for q in range(D // 16):
                    sl = pl.ds(q * 16, 16)
                    ra_v[r, sl] = ra_v[r, sl] + rb_v[r, sl]
                return 0

            lax.fori_loop(0, CCHUNK, add_row, 0)
            pltpu.sync_copy(
                ra_v, out_hbm.at[pl.ds(wid * TOKW + c * CCHUNK, CCHUNK)])

    return k(y_sorted, dest_a2d, dest_b2d)


def kernel(input_batch, probabilities, indices, W1, W2):
    flat_e = indices.reshape(-1).astype(jnp.int32)
    tok = jnp.arange(T * TOPK, dtype=jnp.int32) // TOPK
    onehot = (flat_e[:, None] == jnp.arange(E, dtype=jnp.int32)[None, :])
    onehot = onehot.astype(jnp.int32)
    counts = onehot.sum(axis=0)
    aligned = ((counts + BLK - 1) // BLK) * BLK
    ends = jnp.cumsum(aligned)
    starts = ends - aligned
    csum = jnp.cumsum(onehot, axis=0)
    rank_i = (csum * onehot).sum(axis=1) - 1
    start_i = (starts[None, :] * onehot).sum(axis=1)
    dest = start_i + rank_i
    pad_spread = jnp.arange(NPAD, dtype=jnp.int32) % T
    flat_p = probabilities.reshape(-1).astype(jnp.float32)
    pairs = jnp.stack([tok, lax.bitcast_convert_type(flat_p, jnp.int32)],
                      axis=1)
    init = jnp.stack([pad_spread, jnp.zeros((NPAD,), jnp.int32)], axis=1)
    packed = init.at[dest].set(pairs)
    src_token = packed[:, 0]
    row_prob = lax.bitcast_convert_type(packed[:, 1], jnp.float32)
    block_base = jnp.arange(NB, dtype=jnp.int32) * BLK
    block_expert = jnp.minimum(
        (block_base[:, None] >= ends[None, :]).sum(axis=1), E - 1
    ).astype(jnp.int32)
    block_active = (block_base < ends[E - 1]).astype(jnp.int32)

    x_sorted = _sc_gather(src_token.reshape(NW, -1, GCHUNK), input_batch)
    y_sorted = _tc_expert_mlp(block_expert, block_active, x_sorted,
                              row_prob[:, None], W1, W2)
    d2 = dest.reshape(T, TOPK)
    out = _sc_combine(
        y_sorted,
        d2[:, 0].reshape(NW, -1, CCHUNK),
        d2[:, 1].reshape(NW, -1, CCHUNK))

    total_loss = jnp.zeros((), dtype=jnp.float32)
    return (out, total_loss)
